# Initial kernel scaffold; baseline (speedup 1.0000x reference)
import jax
import jax.numpy as jnp
from jax.experimental import pallas as pl


def kernel(feat, edge_index, W, b, prelu_a):
    raise NotImplementedError("write your pallas kernel here")


# speedup vs baseline: 3.5326x; 3.5326x over previous
import functools

import jax
import jax.numpy as jnp
from jax import lax
from jax.experimental import pallas as pl
from jax.experimental.pallas import tpu as pltpu
from jax.experimental.pallas import tpu_sc as plsc

N = 10000
E = 320000
D = 128
NP = 10240
NC = 2
NS = 16
NW = NC * NS
CH = 128
NCHUNK = 80
EPT = NCHUNK * CH
EP = NW * EPT
ROWS = NP // NS
BLK = 512
NBLK = NP // BLK

_mesh = plsc.VectorSubcoreMesh(core_axis_name="c", subcore_axis_name="s")


@functools.partial(
    pl.kernel,
    out_type=(
        jax.ShapeDtypeStruct((NW, NP), jnp.float32),
        jax.ShapeDtypeStruct((NW, NP), jnp.float32),
    ),
    mesh=_mesh,
    scratch_types=[
        pltpu.VMEM((NCHUNK, CH), jnp.int32),
        pltpu.VMEM((NP,), jnp.float32),
        pltpu.VMEM((NP,), jnp.float32),
    ],
    compiler_params=pltpu.CompilerParams(needs_layout_passes=False),
)
def _hist(src3, dst3, zN, degs_out, degd_out, idx_v, cs, cd):
    c = lax.axis_index("c")
    s = lax.axis_index("s")
    g = s * NC + c
    pltpu.sync_copy(zN, cs)
    pltpu.sync_copy(zN, cd)
    ones = jnp.ones((16,), jnp.float32)

    def count_into(cnt):
        def body(j, carry):
            def inner(k, carry2):
                iv = idx_v[j, pl.ds(k * 16, 16)]
                plsc.addupdate_scatter(cnt, [iv], ones)
                return carry2
            return lax.fori_loop(0, CH // 16, inner, carry)
        lax.fori_loop(0, NCHUNK, body, 0)

    pltpu.sync_copy(src3.at[g], idx_v)
    count_into(cs)
    pltpu.sync_copy(dst3.at[g], idx_v)
    count_into(cd)

    pltpu.sync_copy(cs, degs_out.at[g])
    pltpu.sync_copy(cd, degd_out.at[g])


def _xw_body(feat_ref, degs_ref, w_ref, x_ref):
    d = jnp.sum(degs_ref[...], axis=0)[:, None]
    ns = jnp.where(d > 0, lax.rsqrt(d), 0.0)
    x_ref[...] = jnp.dot(feat_ref[...] * ns, w_ref[...],
                         preferred_element_type=jnp.float32)


def _xw(featp, degs, W):
    return pl.pallas_call(
        _xw_body,
        grid=(NBLK,),
        in_specs=[
            pl.BlockSpec((BLK, D), lambda i: (i, 0)),
            pl.BlockSpec((NW, BLK), lambda i: (0, i)),
            pl.BlockSpec((D, D), lambda i: (0, 0)),
        ],
        out_specs=pl.BlockSpec((BLK, D), lambda i: (i, 0)),
        out_shape=jax.ShapeDtypeStruct((NP, D), jnp.float32),
    )(featp, degs, W)


HALF = NCHUNK // 2


@functools.partial(
    pl.kernel,
    out_type=jax.ShapeDtypeStruct((NC, NP, D), jnp.float32),
    mesh=_mesh,
    scratch_types=[
        pltpu.VMEM((HALF, CH), jnp.int32),
        pltpu.VMEM((HALF, CH), jnp.int32),
        pltpu.VMEM((CH, D), jnp.float32),
        pltpu.VMEM((CH, D), jnp.float32),
        pltpu.VMEM_SHARED((NP, D), jnp.float32),
        pltpu.SemaphoreType.DMA,
        pltpu.SemaphoreType.DMA,
    ],
)
def _agg(x, src3, dst3, out,
         sidx, didx, buf0, buf1, agg_sh, sem0, sem1):
    c = lax.axis_index("c")
    s = lax.axis_index("s")
    g = s * NC + c
    sl = pl.ds(s * ROWS, ROWS)
    pltpu.sync_copy(x.at[pl.ds(N, CH)], buf0)
    for k in range(ROWS // CH):
        pltpu.sync_copy(buf0, agg_sh.at[pl.ds(s * ROWS + k * CH, CH)])
    plsc.subcore_barrier()

    for p in range(2):
        pltpu.sync_copy(src3.at[g, pl.ds(p * HALF, HALF)], sidx)
        pltpu.sync_copy(dst3.at[g, pl.ds(p * HALF, HALF)], didx)

        pltpu.async_copy(x.at[sidx.at[0]], buf0, sem0)

        def body(t, carry):
            j0 = 2 * t
            pltpu.async_copy(x.at[sidx.at[j0 + 1]], buf1, sem1)
            pltpu.make_async_copy(x.at[sidx.at[j0]], buf0, sem0).wait()
            pltpu.sync_copy(buf0, agg_sh.at[didx.at[j0]], add=True)
            pltpu.async_copy(x.at[sidx.at[j0 + 2]], buf0, sem0)
            pltpu.make_async_copy(x.at[sidx.at[j0 + 1]], buf1, sem1).wait()
            pltpu.sync_copy(buf1, agg_sh.at[didx.at[j0 + 1]], add=True)
            return carry

        lax.fori_loop(0, HALF // 2 - 1, body, 0)

        jl = HALF - 2
        pltpu.async_copy(x.at[sidx.at[jl + 1]], buf1, sem1)
        pltpu.make_async_copy(x.at[sidx.at[jl]], buf0, sem0).wait()
        pltpu.sync_copy(buf0, agg_sh.at[didx.at[jl]], add=True)
        pltpu.make_async_copy(x.at[sidx.at[jl + 1]], buf1, sem1).wait()
        pltpu.sync_copy(buf1, agg_sh.at[didx.at[jl + 1]], add=True)

    plsc.subcore_barrier()
    for k in range(ROWS // CH):
        so = s * ROWS + k * CH
        pltpu.sync_copy(agg_sh.at[pl.ds(so, CH)], out.at[c, pl.ds(so, CH)])


def _final_body(aggp_ref, degd_ref, b_ref, a_ref, h_ref, hg_ref):
    i = pl.program_id(0)
    agg = aggp_ref[0] + aggp_ref[1]
    d = jnp.sum(degd_ref[...], axis=0)[:, None]
    nd = jnp.where(d > 0, lax.rsqrt(d), 0.0)
    rst = agg * nd + b_ref[...]
    a = a_ref[0, 0]
    h = jnp.maximum(rst, 0.0) + a * jnp.minimum(rst, 0.0)
    row = i * BLK + lax.broadcasted_iota(jnp.int32, (BLK, 1), 0)
    h = jnp.where(row < N, h, 0.0)
    h_ref[...] = h

    @pl.when(i == 0)
    def _init():
        hg_ref[...] = jnp.zeros_like(hg_ref)

    hg_ref[...] += jnp.sum(h, axis=0, keepdims=True)


def _final(aggp, degd, b2, a2):
    return pl.pallas_call(
        _final_body,
        grid=(NBLK,),
        in_specs=[
            pl.BlockSpec((NC, BLK, D), lambda i: (0, i, 0)),
            pl.BlockSpec((NW, BLK), lambda i: (0, i)),
            pl.BlockSpec((1, D), lambda i: (0, 0)),
            pl.BlockSpec((1, 1), lambda i: (0, 0)),
        ],
        out_specs=[
            pl.BlockSpec((BLK, D), lambda i: (i, 0)),
            pl.BlockSpec((1, D), lambda i: (0, 0)),
        ],
        out_shape=[
            jax.ShapeDtypeStruct((NP, D), jnp.float32),
            jax.ShapeDtypeStruct((1, D), jnp.float32),
        ],
    )(aggp, degd, b2, a2)


def kernel(feat, edge_index, W, b, prelu_a):
    src = edge_index[0].astype(jnp.int32)
    dst = edge_index[1].astype(jnp.int32)
    fill = jnp.full((EP - E,), N, jnp.int32)
    src3 = jnp.concatenate([src, fill]).reshape(NW, NCHUNK, CH)
    dst3 = jnp.concatenate([dst, fill]).reshape(NW, NCHUNK, CH)
    featp = jnp.pad(feat, ((0, NP - N), (0, 0)))

    zN = jnp.zeros((NP,), jnp.float32)

    degs, degd = _hist(src3, dst3, zN)
    x = _xw(featp, degs, W)
    aggp = _agg(x, src3, dst3)
    h_pad, hg = _final(aggp, degd, b.reshape(1, D),
                       jnp.reshape(prelu_a, (1, 1)))
    return h_pad[:N], hg
